# baseline (speedup 1.0000x reference)
import functools
import math

import jax
import jax.numpy as jnp
from jax import lax
from jax.experimental import pallas as pl
from jax.experimental.pallas import tpu as pltpu
from jax.experimental.pallas import tpu_sc as plsc

VOCAB = 100000
EMB = 1024
FF = 256
MAX_LEN = 5000
B, S = 4, 2048
NTOK = B * S
SCALE = math.sqrt(EMB)

_NC, _NS = 2, 16
_NW = _NC * _NS
_B_PER_W = NTOK // _NW
_CHUNK = 64
_NCHUNK = _B_PER_W // _CHUNK


@functools.cache
def _make_gather():
    mesh = plsc.VectorSubcoreMesh(core_axis_name="c", subcore_axis_name="s")

    @functools.partial(
        pl.kernel,
        mesh=mesh,
        out_type=jax.ShapeDtypeStruct((NTOK, EMB), jnp.float32),
        scratch_types=[
            pltpu.VMEM((_NCHUNK, _CHUNK), jnp.int32),
            pltpu.VMEM((_CHUNK, EMB), jnp.float32),
            pltpu.SemaphoreType.DMA,
        ],
    )
    def gather_k(table_hbm, idx_hbm, out_hbm, idx_v, rows_v, sem):
        wid = lax.axis_index("s") * _NC + lax.axis_index("c")
        pltpu.sync_copy(idx_hbm.at[wid], idx_v)
        base = wid * _B_PER_W
        for c in range(_NCHUNK):
            pltpu.async_copy(table_hbm.at[idx_v.at[c]], rows_v, sem).wait()
            pltpu.sync_copy(rows_v, out_hbm.at[pl.ds(base + c * _CHUNK, _CHUNK)])

    return gather_k


_T = 512


def _adapter_body(emb_ref, w1_ref, b1_ref, w2_ref, b2_ref, pe_ref, out_ref):
    e = emb_ref[...]
    h = jnp.maximum(
        jnp.dot(e, w1_ref[...], preferred_element_type=jnp.float32) + b1_ref[...],
        0.0,
    )
    o = e + jnp.dot(h, w2_ref[...], preferred_element_type=jnp.float32) + b2_ref[...]
    out_ref[...] = o * SCALE + pe_ref[...]


def _adapter(emb, W1, b1, W2, b2, pe):
    grid = (NTOK // _T,)
    return pl.pallas_call(
        _adapter_body,
        grid=grid,
        in_specs=[
            pl.BlockSpec((_T, EMB), lambda i: (i, 0)),
            pl.BlockSpec((EMB, FF), lambda i: (0, 0)),
            pl.BlockSpec((1, FF), lambda i: (0, 0)),
            pl.BlockSpec((FF, EMB), lambda i: (0, 0)),
            pl.BlockSpec((1, EMB), lambda i: (0, 0)),
            pl.BlockSpec((_T, EMB), lambda i: (i % (S // _T), 0)),
        ],
        out_specs=pl.BlockSpec((_T, EMB), lambda i: (i, 0)),
        out_shape=jax.ShapeDtypeStruct((NTOK, EMB), jnp.float32),
    )(emb, W1, b1, W2, b2, pe)


def _make_pe():
    pos = jnp.arange(S, dtype=jnp.float32)[:, None]
    div = jnp.exp(
        jnp.arange(0, EMB, 2, dtype=jnp.float32) * (-(math.log(10000.0) / EMB))
    )
    pe = jnp.zeros((S, EMB), dtype=jnp.float32)
    pe = pe.at[:, 0::2].set(jnp.sin(pos * div))
    pe = pe.at[:, 1::2].set(jnp.cos(pos * div))
    return pe


def kernel(x, table, W1, b1, W2, b2):
    idx = x.reshape(_NW, _NCHUNK, _CHUNK).astype(jnp.int32)
    emb = lax.slice(table, (0, 0), (NTOK, EMB))
    pe = _make_pe()
    out = _adapter(emb, W1, b1.reshape(1, FF), W2, b2.reshape(1, EMB), pe)
    return out.reshape(B, S, EMB)


# speedup vs baseline: 1.0587x; 1.0587x over previous
import functools
import math

import jax
import jax.numpy as jnp
from jax import lax
from jax.experimental import pallas as pl
from jax.experimental.pallas import tpu as pltpu
from jax.experimental.pallas import tpu_sc as plsc

VOCAB = 100000
EMB = 1024
FF = 256
MAX_LEN = 5000
B, S = 4, 2048
NTOK = B * S
SCALE = math.sqrt(EMB)

_NC, _NS = 2, 16
_NW = _NC * _NS
_B_PER_W = NTOK // _NW
_CHUNK = 64
_NCHUNK = _B_PER_W // _CHUNK


@functools.cache
def _make_gather():
    mesh = plsc.VectorSubcoreMesh(core_axis_name="c", subcore_axis_name="s")

    @functools.partial(
        pl.kernel,
        mesh=mesh,
        out_type=jax.ShapeDtypeStruct((NTOK, EMB), jnp.float32),
        scratch_types=[
            pltpu.VMEM((_NCHUNK, _CHUNK), jnp.int32),
            pltpu.VMEM((_CHUNK, EMB), jnp.float32),
            pltpu.SemaphoreType.DMA,
        ],
    )
    def gather_k(table_hbm, idx_hbm, out_hbm, idx_v, rows_v, sem):
        wid = lax.axis_index("s") * _NC + lax.axis_index("c")
        pltpu.sync_copy(idx_hbm.at[wid], idx_v)
        base = wid * _B_PER_W
        for c in range(_NCHUNK):
            pltpu.async_copy(table_hbm.at[idx_v.at[c]], rows_v, sem).wait()
            pltpu.sync_copy(rows_v, out_hbm.at[pl.ds(base + c * _CHUNK, _CHUNK)])

    return gather_k


_T = 512


def _adapter_body(emb_ref, w1_ref, b1_ref, w2_ref, b2_ref, pe_ref, out_ref):
    e = emb_ref[0]
    h = jnp.maximum(
        jnp.dot(e, w1_ref[...], preferred_element_type=jnp.float32) + b1_ref[...],
        0.0,
    )
    o = e + jnp.dot(h, w2_ref[...], preferred_element_type=jnp.float32) + b2_ref[...]
    out_ref[0] = o * SCALE + pe_ref[...]


def _adapter(emb, W1, b1, W2, b2, pe):
    emb3 = emb.reshape(B, S, EMB)
    grid = (S // _T, B)
    out = pl.pallas_call(
        _adapter_body,
        grid=grid,
        in_specs=[
            pl.BlockSpec((1, _T, EMB), lambda p, b: (b, p, 0)),
            pl.BlockSpec((EMB, FF), lambda p, b: (0, 0)),
            pl.BlockSpec((1, FF), lambda p, b: (0, 0)),
            pl.BlockSpec((FF, EMB), lambda p, b: (0, 0)),
            pl.BlockSpec((1, EMB), lambda p, b: (0, 0)),
            pl.BlockSpec((_T, EMB), lambda p, b: (p, 0)),
        ],
        out_specs=pl.BlockSpec((1, _T, EMB), lambda p, b: (b, p, 0)),
        out_shape=jax.ShapeDtypeStruct((B, S, EMB), jnp.float32),
    )(emb3, W1, b1, W2, b2, pe)
    return out


def _make_pe():
    pos = jnp.arange(S, dtype=jnp.float32)[:, None]
    div = jnp.exp(
        jnp.arange(0, EMB, 2, dtype=jnp.float32) * (-(math.log(10000.0) / EMB))
    )
    pe = jnp.zeros((S, EMB), dtype=jnp.float32)
    pe = pe.at[:, 0::2].set(jnp.sin(pos * div))
    pe = pe.at[:, 1::2].set(jnp.cos(pos * div))
    return pe


def kernel(x, table, W1, b1, W2, b2):
    idx = x.reshape(_NW, _NCHUNK, _CHUNK).astype(jnp.int32)
    emb = _make_gather()(table, idx)
    pe = _make_pe()
    return _adapter(emb, W1, b1.reshape(1, FF), W2, b2.reshape(1, EMB), pe)


# speedup vs baseline: 1.5120x; 1.4282x over previous
import functools
import math

import numpy as np

import jax
import jax.numpy as jnp
from jax import lax
from jax.experimental import pallas as pl
from jax.experimental.pallas import tpu as pltpu
from jax.experimental.pallas import tpu_sc as plsc

VOCAB = 100000
EMB = 1024
FF = 256
MAX_LEN = 5000
B, S = 4, 2048
NTOK = B * S
SCALE = math.sqrt(EMB)

_NC, _NS = 2, 16
_NW = _NC * _NS
_B_PER_W = NTOK // _NW
_CHUNK = 32
_NCHUNK = _B_PER_W // _CHUNK


@functools.cache
def _make_gather():
    mesh = plsc.VectorSubcoreMesh(core_axis_name="c", subcore_axis_name="s")

    @functools.partial(
        pl.kernel,
        mesh=mesh,
        out_type=jax.ShapeDtypeStruct((NTOK, EMB), jnp.float32),
        scratch_types=[
            pltpu.VMEM((_NCHUNK, _CHUNK), jnp.int32),
            pltpu.VMEM((2, _CHUNK, EMB), jnp.float32),
            pltpu.SemaphoreType.DMA,
            pltpu.SemaphoreType.DMA,
            pltpu.SemaphoreType.DMA,
            pltpu.SemaphoreType.DMA,
        ],
    )
    def gather_k(table_hbm, idx_hbm, out_hbm, idx_v, rows_v, g0, g1, w0, w1):
        wid = lax.axis_index("s") * _NC + lax.axis_index("c")
        pltpu.sync_copy(idx_hbm.at[wid], idx_v)
        base = wid * _B_PER_W
        gsem = (g0, g1)
        wsem = (w0, w1)

        def out_slice(c):
            return out_hbm.at[pl.ds(base + c * _CHUNK, _CHUNK)]

        pltpu.async_copy(table_hbm.at[idx_v.at[0]], rows_v.at[0], gsem[0])
        for c in range(_NCHUNK):
            s = c % 2
            if c + 1 < _NCHUNK:
                s2 = (c + 1) % 2
                if c >= 1:
                    pltpu.make_async_copy(rows_v.at[s2], out_slice(c - 1),
                                          wsem[s2]).wait()
                pltpu.async_copy(table_hbm.at[idx_v.at[c + 1]], rows_v.at[s2],
                                 gsem[s2])
            pltpu.make_async_copy(table_hbm.at[idx_v.at[c]], rows_v.at[s],
                                  gsem[s]).wait()
            pltpu.async_copy(rows_v.at[s], out_slice(c), wsem[s])
        pltpu.make_async_copy(rows_v.at[(_NCHUNK - 2) % 2],
                              out_slice(_NCHUNK - 2),
                              wsem[(_NCHUNK - 2) % 2]).wait()
        pltpu.make_async_copy(rows_v.at[(_NCHUNK - 1) % 2],
                              out_slice(_NCHUNK - 1),
                              wsem[(_NCHUNK - 1) % 2]).wait()

    return gather_k


_T = 512


def _adapter_body(emb_ref, w1_ref, b1_ref, w2_ref, b2_ref, sint_ref,
                  cost_ref, sinb_ref, cosb_ref, out_ref, pe_s):
    b = pl.program_id(1)

    @pl.when(b == 0)
    def _():
        pe_s[...] = (sinb_ref[0] * cost_ref[...]
                     + cosb_ref[0] * sint_ref[...])

    e = emb_ref[0]
    h = jnp.maximum(
        jnp.dot(e, w1_ref[...], preferred_element_type=jnp.float32) + b1_ref[...],
        0.0,
    )
    o = e + jnp.dot(h, w2_ref[...], preferred_element_type=jnp.float32) + b2_ref[...]
    out_ref[0] = o * SCALE + pe_s[...]


def _adapter(emb, W1, b1, W2, b2, sinT, cosT, sinB, cosB):
    emb3 = emb.reshape(B, S, EMB)
    grid = (S // _T, B)
    return pl.pallas_call(
        _adapter_body,
        grid=grid,
        in_specs=[
            pl.BlockSpec((1, _T, EMB), lambda p, b: (b, p, 0)),
            pl.BlockSpec((EMB, FF), lambda p, b: (0, 0)),
            pl.BlockSpec((1, FF), lambda p, b: (0, 0)),
            pl.BlockSpec((FF, EMB), lambda p, b: (0, 0)),
            pl.BlockSpec((1, EMB), lambda p, b: (0, 0)),
            pl.BlockSpec((_T, EMB), lambda p, b: (0, 0)),
            pl.BlockSpec((_T, EMB), lambda p, b: (0, 0)),
            pl.BlockSpec((1, 1, EMB), lambda p, b: (p, 0, 0)),
            pl.BlockSpec((1, 1, EMB), lambda p, b: (p, 0, 0)),
        ],
        out_specs=pl.BlockSpec((1, _T, EMB), lambda p, b: (b, p, 0)),
        out_shape=jax.ShapeDtypeStruct((B, S, EMB), jnp.float32),
        scratch_shapes=[pltpu.VMEM((_T, EMB), jnp.float32)],
    )(emb3, W1, b1, W2, b2, sinT, cosT, sinB, cosB)


def _pe_consts():
    half = np.exp(np.arange(0, EMB, 2, dtype=np.float64)
                  * (-(math.log(10000.0) / EMB)))
    divf = np.repeat(half, 2)
    phase = np.tile(np.array([0.0, math.pi / 2]), EMB // 2)
    t = np.arange(_T, dtype=np.float64)[:, None]
    sinT = np.sin(t * divf + phase).astype(np.float32)
    cosT = np.cos(t * divf + phase).astype(np.float32)
    p = np.arange(S // _T, dtype=np.float64)[:, None]
    sinB = np.sin(p * _T * divf).astype(np.float32)
    cosB = np.cos(p * _T * divf).astype(np.float32)
    return (jnp.asarray(sinT), jnp.asarray(cosT),
            jnp.asarray(sinB[:, None, :]), jnp.asarray(cosB[:, None, :]))


def kernel(x, table, W1, b1, W2, b2):
    idx = x.reshape(_NW, _NCHUNK, _CHUNK).astype(jnp.int32)
    emb = _make_gather()(table, idx)
    sinT, cosT, sinB, cosB = _pe_consts()
    return _adapter(emb, W1, b1.reshape(1, FF), W2, b2.reshape(1, EMB),
                    sinT, cosT, sinB, cosB)
